# Initial kernel scaffold; baseline (speedup 1.0000x reference)
import jax
import jax.numpy as jnp
from jax.experimental import pallas as pl


def kernel(x, pos, edge_index, params):
    raise NotImplementedError("write your pallas kernel here")


# speedup vs baseline: 1.6992x; 1.6992x over previous
import functools

import jax
import jax.numpy as jnp
from jax import lax
from jax.experimental import pallas as pl
from jax.experimental.pallas import tpu as pltpu
from jax.experimental.pallas import tpu_sc as plsc

N = 10000
E = 160000
D = 256
Q = 4
DQ = 64
H = 64

NC = 2
NS = 16
NW = NC * NS

ECH = E // NW
EC = E // NS
BLK = 80
NBLK = EC // BLK
NP = 10240
ROWS_PER = NP // NS

NB = 400
EB = 640

_mesh = plsc.VectorSubcoreMesh(core_axis_name="c", subcore_axis_name="s")
_sc_params = pltpu.CompilerParams(needs_layout_passes=False,
                                  use_tc_tiling_on_sc=False)


def _d2_body(pos_hbm, src_hbm, dst_hbm, d2_hbm, pos_v, src_v, dst_v, out_v):
    c = lax.axis_index("c")
    s = lax.axis_index("s")
    wid = s * NC + c
    base = wid * ECH
    pltpu.sync_copy(pos_hbm, pos_v)
    pltpu.sync_copy(src_hbm.at[pl.ds(base, ECH)], src_v)
    pltpu.sync_copy(dst_hbm.at[pl.ds(base, ECH)], dst_v)

    def body(i, carry):
        off = jnp.minimum(i * 16, ECH - 16)
        si = src_v[pl.ds(off, 16)] * 8
        di = dst_v[pl.ds(off, 16)] * 8
        acc = jnp.zeros((16,), jnp.float32)
        for cdim in range(3):
            a = plsc.load_gather(pos_v, [si + cdim])
            b = plsc.load_gather(pos_v, [di + cdim])
            dd = b - a
            acc = acc + dd * dd
        out_v[pl.ds(off, 16)] = acc
        return carry

    lax.fori_loop(0, (ECH + 15) // 16, body, 0)
    pltpu.sync_copy(out_v, d2_hbm.at[pl.ds(base, ECH)])


_d2_call = pl.kernel(
    _d2_body,
    out_type=jax.ShapeDtypeStruct((E,), jnp.float32),
    mesh=_mesh,
    scratch_types=[
        pltpu.VMEM((N * 8,), jnp.float32),
        pltpu.VMEM((ECH,), jnp.int32),
        pltpu.VMEM((ECH,), jnp.int32),
        pltpu.VMEM((ECH,), jnp.float32),
    ],
    compiler_params=_sc_params,
)


def _radial_body(d2_ref, w1_ref, b1_ref, w2_ref, b2_ref, w3_ref, b3_ref, o_ref):
    d2 = d2_ref[...]
    dist = jnp.sqrt(d2 + 1e-12)
    h = jnp.maximum(dist * w1_ref[...] + b1_ref[...], 0.0)
    h = jnp.maximum(
        jnp.dot(h, w2_ref[...], preferred_element_type=jnp.float32)
        + b2_ref[...], 0.0)
    w = jnp.dot(h, w3_ref[...], preferred_element_type=jnp.float32) + b3_ref[...]
    w = w * 0.25
    for q in range(Q):
        o_ref[q] = w[:, q * DQ:(q + 1) * DQ]


_radial_call = pl.pallas_call(
    _radial_body,
    grid=(E // EB,),
    in_specs=[
        pl.BlockSpec((EB, 1), lambda i: (i, 0)),
        pl.BlockSpec((1, H), lambda i: (0, 0)),
        pl.BlockSpec((1, H), lambda i: (0, 0)),
        pl.BlockSpec((H, H), lambda i: (0, 0)),
        pl.BlockSpec((1, H), lambda i: (0, 0)),
        pl.BlockSpec((H, D), lambda i: (0, 0)),
        pl.BlockSpec((1, D), lambda i: (0, 0)),
    ],
    out_specs=pl.BlockSpec((Q, EB, DQ), lambda i: (0, i, 0)),
    out_shape=jax.ShapeDtypeStruct((Q, E, DQ), jnp.float32),
)


def _ln_mm_math(x, g, b, wm, ws, bs):
    mu = jnp.mean(x, axis=1, keepdims=True)
    var = jnp.mean((x - mu) * (x - mu), axis=1, keepdims=True)
    h = (x - mu) / jnp.sqrt(var + 1e-5) * g + b
    xm = jnp.dot(h, wm, preferred_element_type=jnp.float32)
    xs = jnp.dot(h, ws, preferred_element_type=jnp.float32) + bs
    return xm, xs


def _ln_mm_body(x_ref, g_ref, b_ref, wm_ref, ws_ref, bs_ref, xm_ref, xs_ref):
    xm, xs = _ln_mm_math(x_ref[...], g_ref[...], b_ref[...],
                         wm_ref[...], ws_ref[...], bs_ref[...])
    for q in range(Q):
        xm_ref[q] = xm[:, q * DQ:(q + 1) * DQ]
    xs_ref[...] = xs


def _combine_ln_mm_body(xs_in_ref, agg_ref, skip_ref, g_ref, b_ref,
                        wm_ref, ws_ref, bs_ref, feat_ref, xm_ref, xs_ref):
    a = agg_ref[...]
    conv = xs_in_ref[...] + jnp.concatenate([a[q] for q in range(Q)], axis=1)
    feat = jnp.maximum(conv, 0.0) + skip_ref[...]
    feat_ref[...] = feat
    xm, xs = _ln_mm_math(feat, g_ref[...], b_ref[...],
                         wm_ref[...], ws_ref[...], bs_ref[...])
    for q in range(Q):
        xm_ref[q] = xm[:, q * DQ:(q + 1) * DQ]
    xs_ref[...] = xs


def _final_body(xs_in_ref, agg_ref, g_ref, b_ref, o_ref):
    a = agg_ref[...]
    x = xs_in_ref[...] + jnp.concatenate([a[q] for q in range(Q)], axis=1)
    mu = jnp.mean(x, axis=1, keepdims=True)
    var = jnp.mean((x - mu) * (x - mu), axis=1, keepdims=True)
    o_ref[...] = (x - mu) / jnp.sqrt(var + 1e-5) * g_ref[...] + b_ref[...]


_vec_spec = pl.BlockSpec((1, D), lambda i: (0, 0))
_mat_spec = pl.BlockSpec((D, D), lambda i: (0, 0))
_row_spec = pl.BlockSpec((NB, D), lambda i: (i, 0))
_quarter_spec = pl.BlockSpec((Q, NB, DQ), lambda i: (0, i, 0))

_ln_mm_call = pl.pallas_call(
    _ln_mm_body,
    grid=(N // NB,),
    in_specs=[_row_spec, _vec_spec, _vec_spec, _mat_spec, _mat_spec, _vec_spec],
    out_specs=[_quarter_spec, _row_spec],
    out_shape=[jax.ShapeDtypeStruct((Q, N, DQ), jnp.float32),
               jax.ShapeDtypeStruct((N, D), jnp.float32)],
)

_combine_ln_mm_call = pl.pallas_call(
    _combine_ln_mm_body,
    grid=(N // NB,),
    in_specs=[_row_spec, _quarter_spec, _row_spec,
              _vec_spec, _vec_spec, _mat_spec, _mat_spec, _vec_spec],
    out_specs=[_row_spec, _quarter_spec, _row_spec],
    out_shape=[jax.ShapeDtypeStruct((N, D), jnp.float32),
               jax.ShapeDtypeStruct((Q, N, DQ), jnp.float32),
               jax.ShapeDtypeStruct((N, D), jnp.float32)],
)

_final_call = pl.pallas_call(
    _final_body,
    grid=(N // NB,),
    in_specs=[_row_spec, _quarter_spec, _vec_spec, _vec_spec],
    out_specs=_row_spec,
    out_shape=jax.ShapeDtypeStruct((N, D), jnp.float32),
)


def _msg_body(xm_hbm, w_hbm, src3_hbm, dst3_hbm, zero_hbm, out_hbm,
              src_v, dst_v, wbuf, rows, acc, sem):
    c = lax.axis_index("c")
    s = lax.axis_index("s")
    pltpu.sync_copy(src3_hbm.at[s], src_v)
    pltpu.sync_copy(dst3_hbm.at[s], dst_v)

    def shift(delta):
        def off_body(r, carry):
            for k in range(BLK // 16):
                sl = src_v[r, pl.ds(k * 16, 16)]
                src_v[r, pl.ds(k * 16, 16)] = sl + delta
            return carry

        lax.fori_loop(0, NBLK, off_body, 0)

    shift((2 * c * N).astype(jnp.int32))
    for qq in range(2):
        if qq:
            shift(jnp.int32(N))
        f = 2 * c + qq
        pltpu.sync_copy(zero_hbm.at[pl.ds(s * ROWS_PER, ROWS_PER)],
                        acc.at[pl.ds(s * ROWS_PER, ROWS_PER)])
        plsc.subcore_barrier()
        ebase = f * E + s * EC

        def blk_body(j, carry):
            pltpu.sync_copy(w_hbm.at[pl.ds(ebase + j * BLK, BLK)], wbuf)
            pltpu.async_copy(xm_hbm.at[src_v.at[j]], rows, sem).wait()

            def mul_body(r, carry2):
                for k in range(DQ // 16):
                    v = rows[r, pl.ds(k * 16, 16)] * wbuf[r, pl.ds(k * 16, 16)]
                    rows[r, pl.ds(k * 16, 16)] = v
                return carry2

            lax.fori_loop(0, BLK, mul_body, 0)
            pltpu.sync_copy(rows, acc.at[dst_v.at[j]], add=True)
            return carry

        lax.fori_loop(0, NBLK, blk_body, 0)
        plsc.subcore_barrier()
        pltpu.sync_copy(acc.at[pl.ds(s * ROWS_PER, ROWS_PER)],
                        out_hbm.at[pl.ds(f * NP + s * ROWS_PER, ROWS_PER)])


_msg_call = pl.kernel(
    _msg_body,
    out_type=jax.ShapeDtypeStruct((Q * NP, DQ), jnp.float32),
    mesh=_mesh,
    scratch_types=[
        pltpu.VMEM((NBLK, BLK), jnp.int32),
        pltpu.VMEM((NBLK, BLK), jnp.int32),
        pltpu.VMEM((BLK, DQ), jnp.float32),
        pltpu.VMEM((BLK, DQ), jnp.float32),
        pltpu.VMEM_SHARED((NP, DQ), jnp.float32),
        pltpu.SemaphoreType.DMA,
    ],
    compiler_params=_sc_params,
)


def _row(v):
    return v.reshape(1, -1)


def kernel(x, pos, edge_index, params):
    src = edge_index[0]
    dst = edge_index[1]
    pos8 = jnp.concatenate(
        [pos, jnp.zeros((N, 5), jnp.float32)], axis=1).reshape(-1)
    src3 = src.reshape(NS, NBLK, BLK)
    dst3 = dst.reshape(NS, NBLK, BLK)
    zero_q = jnp.zeros((NP, DQ), jnp.float32)
    zero_full = jnp.zeros((N, D), jnp.float32)

    d2 = _d2_call(pos8, src, dst)
    d2c = d2.reshape(E, 1)

    def radial(p):
        return _radial_call(d2c, p["mlp_w1"], _row(p["mlp_b1"]),
                            p["mlp_w2"], _row(p["mlp_b2"]),
                            p["mlp_w3"], _row(p["mlp_b3"]))

    p0 = params["layer0"]
    ps = params["layer_s"]
    p1 = params["layer1"]
    wgt0f = radial(p0).reshape(Q * E, DQ)
    wgtsf = radial(ps).reshape(Q * E, DQ)
    wgt1f = radial(p1).reshape(Q * E, DQ)

    def agg(xm, wf):
        out = _msg_call(xm.reshape(Q * N, DQ), wf, src3, dst3, zero_q)
        return out.reshape(Q, NP, DQ)

    ln0 = params["ln0"]
    ln1 = params["ln1"]
    ln2 = params["ln2"]

    xm, xs = _ln_mm_call(x, _row(ln0["gamma"]), _row(ln0["beta"]),
                         p0["wmsg"], p0["wself"], _row(p0["bself"]))
    a = agg(xm, wgt0f)

    feat = zero_full
    for _ in range(4):
        feat, xm, xs = _combine_ln_mm_call(
            xs, a, feat, _row(ln1["gamma"]), _row(ln1["beta"]),
            ps["wmsg"], ps["wself"], _row(ps["bself"]))
        a = agg(xm, wgtsf)

    feat, xm, xs = _combine_ln_mm_call(
        xs, a, feat, _row(ln1["gamma"]), _row(ln1["beta"]),
        p1["wmsg"], p1["wself"], _row(p1["bself"]))
    a = agg(xm, wgt1f)

    return _final_call(xs, a, _row(ln2["gamma"]), _row(ln2["beta"]))


# speedup vs baseline: 2.7560x; 1.6219x over previous
import functools

import jax
import jax.numpy as jnp
from jax import lax
from jax.experimental import pallas as pl
from jax.experimental.pallas import tpu as pltpu
from jax.experimental.pallas import tpu_sc as plsc

N = 10000
E = 160000
D = 256
Q = 4
DQ = 64
H = 64

NC = 2
NS = 16
NW = NC * NS

ECH = E // NW
EC = E // NS
BLK = 80
NBLK = EC // BLK
NP = 10240
ROWS_PER = NP // NS

NB = 400
EB = 640

_mesh = plsc.VectorSubcoreMesh(core_axis_name="c", subcore_axis_name="s")
_sc_params = pltpu.CompilerParams(needs_layout_passes=False,
                                  use_tc_tiling_on_sc=False)


def _d2_body(pos_hbm, src_hbm, dst_hbm, d2_hbm, pos_v, src_v, dst_v, out_v):
    c = lax.axis_index("c")
    s = lax.axis_index("s")
    wid = s * NC + c
    base = wid * ECH
    pltpu.sync_copy(pos_hbm, pos_v)
    pltpu.sync_copy(src_hbm.at[pl.ds(base, ECH)], src_v)
    pltpu.sync_copy(dst_hbm.at[pl.ds(base, ECH)], dst_v)

    def body(i, carry):
        off = jnp.minimum(i * 16, ECH - 16)
        si = src_v[pl.ds(off, 16)] * 8
        di = dst_v[pl.ds(off, 16)] * 8
        acc = jnp.zeros((16,), jnp.float32)
        for cdim in range(3):
            a = plsc.load_gather(pos_v, [si + cdim])
            b = plsc.load_gather(pos_v, [di + cdim])
            dd = b - a
            acc = acc + dd * dd
        out_v[pl.ds(off, 16)] = acc
        return carry

    lax.fori_loop(0, (ECH + 15) // 16, body, 0)
    pltpu.sync_copy(out_v, d2_hbm.at[pl.ds(base, ECH)])


_d2_call = pl.kernel(
    _d2_body,
    out_type=jax.ShapeDtypeStruct((E,), jnp.float32),
    mesh=_mesh,
    scratch_types=[
        pltpu.VMEM((N * 8,), jnp.float32),
        pltpu.VMEM((ECH,), jnp.int32),
        pltpu.VMEM((ECH,), jnp.int32),
        pltpu.VMEM((ECH,), jnp.float32),
    ],
    compiler_params=_sc_params,
)


def _radial_body(d2_ref, w1_ref, b1_ref, w2_ref, b2_ref, w3_ref, b3_ref, o_ref):
    d2 = d2_ref[...]
    dist = jnp.sqrt(d2 + 1e-12)
    h = jnp.maximum(dist * w1_ref[...] + b1_ref[...], 0.0)
    h = jnp.maximum(
        jnp.dot(h, w2_ref[...], preferred_element_type=jnp.float32)
        + b2_ref[...], 0.0)
    w = jnp.dot(h, w3_ref[...], preferred_element_type=jnp.float32) + b3_ref[...]
    w = w * 0.25
    for q in range(Q):
        o_ref[q] = w[:, q * DQ:(q + 1) * DQ]


_radial_call = pl.pallas_call(
    _radial_body,
    grid=(E // EB,),
    in_specs=[
        pl.BlockSpec((EB, 1), lambda i: (i, 0)),
        pl.BlockSpec((1, H), lambda i: (0, 0)),
        pl.BlockSpec((1, H), lambda i: (0, 0)),
        pl.BlockSpec((H, H), lambda i: (0, 0)),
        pl.BlockSpec((1, H), lambda i: (0, 0)),
        pl.BlockSpec((H, D), lambda i: (0, 0)),
        pl.BlockSpec((1, D), lambda i: (0, 0)),
    ],
    out_specs=pl.BlockSpec((Q, EB, DQ), lambda i: (0, i, 0)),
    out_shape=jax.ShapeDtypeStruct((Q, E, DQ), jnp.float32),
)


def _ln_mm_math(x, g, b, wm, ws, bs):
    mu = jnp.mean(x, axis=1, keepdims=True)
    var = jnp.mean((x - mu) * (x - mu), axis=1, keepdims=True)
    h = (x - mu) / jnp.sqrt(var + 1e-5) * g + b
    xm = jnp.dot(h, wm, preferred_element_type=jnp.float32)
    xs = jnp.dot(h, ws, preferred_element_type=jnp.float32) + bs
    return xm, xs


def _ln_mm_body(x_ref, g_ref, b_ref, wm_ref, ws_ref, bs_ref, xm_ref, xs_ref):
    xm, xs = _ln_mm_math(x_ref[...], g_ref[...], b_ref[...],
                         wm_ref[...], ws_ref[...], bs_ref[...])
    for q in range(Q):
        xm_ref[q] = xm[:, q * DQ:(q + 1) * DQ]
    xs_ref[...] = xs


def _combine_ln_mm_body(xs_in_ref, agg_ref, skip_ref, g_ref, b_ref,
                        wm_ref, ws_ref, bs_ref, feat_ref, xm_ref, xs_ref):
    a = agg_ref[...]
    conv = xs_in_ref[...] + jnp.concatenate([a[q] for q in range(Q)], axis=1)
    feat = jnp.maximum(conv, 0.0) + skip_ref[...]
    feat_ref[...] = feat
    xm, xs = _ln_mm_math(feat, g_ref[...], b_ref[...],
                         wm_ref[...], ws_ref[...], bs_ref[...])
    for q in range(Q):
        xm_ref[q] = xm[:, q * DQ:(q + 1) * DQ]
    xs_ref[...] = xs


def _final_body(xs_in_ref, agg_ref, g_ref, b_ref, o_ref):
    a = agg_ref[...]
    x = xs_in_ref[...] + jnp.concatenate([a[q] for q in range(Q)], axis=1)
    mu = jnp.mean(x, axis=1, keepdims=True)
    var = jnp.mean((x - mu) * (x - mu), axis=1, keepdims=True)
    o_ref[...] = (x - mu) / jnp.sqrt(var + 1e-5) * g_ref[...] + b_ref[...]


_vec_spec = pl.BlockSpec((1, D), lambda i: (0, 0))
_mat_spec = pl.BlockSpec((D, D), lambda i: (0, 0))
_row_spec = pl.BlockSpec((NB, D), lambda i: (i, 0))
_quarter_spec = pl.BlockSpec((Q, NB, DQ), lambda i: (0, i, 0))

_ln_mm_call = pl.pallas_call(
    _ln_mm_body,
    grid=(N // NB,),
    in_specs=[_row_spec, _vec_spec, _vec_spec, _mat_spec, _mat_spec, _vec_spec],
    out_specs=[_quarter_spec, _row_spec],
    out_shape=[jax.ShapeDtypeStruct((Q, N, DQ), jnp.float32),
               jax.ShapeDtypeStruct((N, D), jnp.float32)],
)

_combine_ln_mm_call = pl.pallas_call(
    _combine_ln_mm_body,
    grid=(N // NB,),
    in_specs=[_row_spec, _quarter_spec, _row_spec,
              _vec_spec, _vec_spec, _mat_spec, _mat_spec, _vec_spec],
    out_specs=[_row_spec, _quarter_spec, _row_spec],
    out_shape=[jax.ShapeDtypeStruct((N, D), jnp.float32),
               jax.ShapeDtypeStruct((Q, N, DQ), jnp.float32),
               jax.ShapeDtypeStruct((N, D), jnp.float32)],
)

_final_call = pl.pallas_call(
    _final_body,
    grid=(N // NB,),
    in_specs=[_row_spec, _quarter_spec, _vec_spec, _vec_spec],
    out_specs=_row_spec,
    out_shape=jax.ShapeDtypeStruct((N, D), jnp.float32),
)


def _msg_body(xm_hbm, w_hbm, src3_hbm, dst3_hbm, out_hbm,
              src_v, dst_v, wb_a, wb_b, rows_a, rows_b, prod_a, prod_b,
              zbuf, acc, sem_ga, sem_gb, sem_wa, sem_wb, sem_sa, sem_sb):
    c = lax.axis_index("c")
    s = lax.axis_index("s")
    pltpu.sync_copy(src3_hbm.at[s], src_v)
    pltpu.sync_copy(dst3_hbm.at[s], dst_v)

    def zb_body(r, carry):
        zero16 = jnp.zeros((16,), jnp.float32)
        for rr in range(4):
            for k in range(DQ // 16):
                zbuf[r * 4 + rr, pl.ds(k * 16, 16)] = zero16
        return carry

    lax.fori_loop(0, BLK // 4, zb_body, 0)

    def shift(delta):
        def off_body(r, carry):
            for k in range(BLK // 16):
                sl = src_v[r, pl.ds(k * 16, 16)]
                src_v[r, pl.ds(k * 16, 16)] = sl + delta
            return carry

        lax.fori_loop(0, NBLK, off_body, 0)

    def issue_in(j, rows, wb, sem_g, sem_w, ebase):
        pltpu.async_copy(xm_hbm.at[src_v.at[j]], rows, sem_g)
        pltpu.async_copy(w_hbm.at[pl.ds(ebase + j * BLK, BLK)], wb, sem_w)

    def wait_in(j, rows, wb, sem_g, sem_w, ebase):
        pltpu.make_async_copy(xm_hbm.at[src_v.at[j]], rows, sem_g).wait()
        pltpu.make_async_copy(
            w_hbm.at[pl.ds(ebase + j * BLK, BLK)], wb, sem_w).wait()

    def mul(rows, wb, prod):
        def mul_body(r, carry2):
            for rr in range(4):
                for k in range(DQ // 16):
                    v = (rows[r * 4 + rr, pl.ds(k * 16, 16)]
                         * wb[r * 4 + rr, pl.ds(k * 16, 16)])
                    prod[r * 4 + rr, pl.ds(k * 16, 16)] = v
            return carry2

        lax.fori_loop(0, BLK // 4, mul_body, 0)

    def scat_issue(j, prod, sem_s):
        pltpu.async_copy(prod, acc.at[dst_v.at[j]], sem_s, add=True)

    def scat_wait(j, prod, sem_s):
        pltpu.make_async_copy(prod, acc.at[dst_v.at[j]], sem_s).wait()

    shift((2 * c * N).astype(jnp.int32))
    for qq in range(2):
        if qq:
            shift(jnp.int32(N))
        f = 2 * c + qq
        for z in range(ROWS_PER // BLK):
            pltpu.sync_copy(zbuf, acc.at[pl.ds(s * ROWS_PER + z * BLK, BLK)])
        plsc.subcore_barrier()
        ebase = f * E + s * EC

        issue_in(0, rows_a, wb_a, sem_ga, sem_wa, ebase)

        def pair_body(i, carry):
            ja = 2 * i
            jb = 2 * i + 1

            @pl.when(i > 0)
            def _():
                scat_wait(ja - 2, prod_a, sem_sa)

            issue_in(jb, rows_b, wb_b, sem_gb, sem_wb, ebase)
            wait_in(ja, rows_a, wb_a, sem_ga, sem_wa, ebase)
            mul(rows_a, wb_a, prod_a)
            scat_issue(ja, prod_a, sem_sa)

            @pl.when(i > 0)
            def _():
                scat_wait(jb - 2, prod_b, sem_sb)

            issue_in(jb + 1, rows_a, wb_a, sem_ga, sem_wa, ebase)
            wait_in(jb, rows_b, wb_b, sem_gb, sem_wb, ebase)
            mul(rows_b, wb_b, prod_b)
            scat_issue(jb, prod_b, sem_sb)
            return carry

        lax.fori_loop(0, (NBLK - 1) // 2, pair_body, 0)

        jl = NBLK - 1
        scat_wait(jl - 2, prod_a, sem_sa)
        wait_in(jl, rows_a, wb_a, sem_ga, sem_wa, ebase)
        mul(rows_a, wb_a, prod_a)
        scat_issue(jl, prod_a, sem_sa)
        scat_wait(jl - 1, prod_b, sem_sb)
        scat_wait(jl, prod_a, sem_sa)
        plsc.subcore_barrier()
        pltpu.sync_copy(acc.at[pl.ds(s * ROWS_PER, ROWS_PER)],
                        out_hbm.at[pl.ds(f * NP + s * ROWS_PER, ROWS_PER)])


_msg_call = pl.kernel(
    _msg_body,
    out_type=jax.ShapeDtypeStruct((Q * NP, DQ), jnp.float32),
    mesh=_mesh,
    scratch_types=[
        pltpu.VMEM((NBLK, BLK), jnp.int32),
        pltpu.VMEM((NBLK, BLK), jnp.int32),
        pltpu.VMEM((BLK, DQ), jnp.float32),
        pltpu.VMEM((BLK, DQ), jnp.float32),
        pltpu.VMEM((BLK, DQ), jnp.float32),
        pltpu.VMEM((BLK, DQ), jnp.float32),
        pltpu.VMEM((BLK, DQ), jnp.float32),
        pltpu.VMEM((BLK, DQ), jnp.float32),
        pltpu.VMEM((BLK, DQ), jnp.float32),
        pltpu.VMEM_SHARED((NP, DQ), jnp.float32),
        pltpu.SemaphoreType.DMA,
        pltpu.SemaphoreType.DMA,
        pltpu.SemaphoreType.DMA,
        pltpu.SemaphoreType.DMA,
        pltpu.SemaphoreType.DMA,
        pltpu.SemaphoreType.DMA,
    ],
    compiler_params=_sc_params,
)


def _row(v):
    return v.reshape(1, -1)


def kernel(x, pos, edge_index, params):
    src = edge_index[0]
    dst = edge_index[1]
    pos8 = jnp.concatenate(
        [pos, jnp.zeros((N, 5), jnp.float32)], axis=1).reshape(-1)
    src3 = src.reshape(NS, NBLK, BLK)
    dst3 = dst.reshape(NS, NBLK, BLK)
    zero_full = jnp.zeros((N, D), jnp.float32)

    d2 = _d2_call(pos8, src, dst)
    d2c = d2.reshape(E, 1)

    def radial(p):
        return _radial_call(d2c, p["mlp_w1"], _row(p["mlp_b1"]),
                            p["mlp_w2"], _row(p["mlp_b2"]),
                            p["mlp_w3"], _row(p["mlp_b3"]))

    p0 = params["layer0"]
    ps = params["layer_s"]
    p1 = params["layer1"]
    wgt0f = radial(p0).reshape(Q * E, DQ)
    wgtsf = radial(ps).reshape(Q * E, DQ)
    wgt1f = radial(p1).reshape(Q * E, DQ)

    def agg(xm, wf):
        out = _msg_call(xm.reshape(Q * N, DQ), wf, src3, dst3)
        return out.reshape(Q, NP, DQ)

    ln0 = params["ln0"]
    ln1 = params["ln1"]
    ln2 = params["ln2"]

    xm, xs = _ln_mm_call(x, _row(ln0["gamma"]), _row(ln0["beta"]),
                         p0["wmsg"], p0["wself"], _row(p0["bself"]))
    a = agg(xm, wgt0f)

    feat = zero_full
    for _ in range(4):
        feat, xm, xs = _combine_ln_mm_call(
            xs, a, feat, _row(ln1["gamma"]), _row(ln1["beta"]),
            ps["wmsg"], ps["wself"], _row(ps["bself"]))
        a = agg(xm, wgtsf)

    feat, xm, xs = _combine_ln_mm_call(
        xs, a, feat, _row(ln1["gamma"]), _row(ln1["beta"]),
        p1["wmsg"], p1["wself"], _row(p1["bself"]))
    a = agg(xm, wgt1f)

    return _final_call(xs, a, _row(ln2["gamma"]), _row(ln2["beta"]))


# speedup vs baseline: 3.1167x; 1.1309x over previous
import functools

import jax
import jax.numpy as jnp
from jax import lax
from jax.experimental import pallas as pl
from jax.experimental.pallas import tpu as pltpu
from jax.experimental.pallas import tpu_sc as plsc

N = 10000
E = 160000
D = 256
Q = 4
DQ = 64
H = 64

NC = 2
NS = 16
NW = NC * NS

ECH = E // NW
EC = E // NS
BLK = 80
NBLK = EC // BLK
NP = 10240
ROWS_PER = NP // NS

NB = 1000
EB = 3200

_mesh = plsc.VectorSubcoreMesh(core_axis_name="c", subcore_axis_name="s")
_sc_params = pltpu.CompilerParams(needs_layout_passes=False,
                                  use_tc_tiling_on_sc=False)


def _d2_body(pos_hbm, src_hbm, dst_hbm, d2_hbm, pos_v, src_v, dst_v, out_v):
    c = lax.axis_index("c")
    s = lax.axis_index("s")
    wid = s * NC + c
    base = wid * ECH
    pltpu.sync_copy(pos_hbm, pos_v)
    pltpu.sync_copy(src_hbm.at[pl.ds(base, ECH)], src_v)
    pltpu.sync_copy(dst_hbm.at[pl.ds(base, ECH)], dst_v)

    def body(i, carry):
        off = jnp.minimum(i * 16, ECH - 16)
        si = src_v[pl.ds(off, 16)] * 8
        di = dst_v[pl.ds(off, 16)] * 8
        acc = jnp.zeros((16,), jnp.float32)
        for cdim in range(3):
            a = plsc.load_gather(pos_v, [si + cdim])
            b = plsc.load_gather(pos_v, [di + cdim])
            dd = b - a
            acc = acc + dd * dd
        out_v[pl.ds(off, 16)] = acc
        return carry

    lax.fori_loop(0, (ECH + 15) // 16, body, 0)
    pltpu.sync_copy(out_v, d2_hbm.at[pl.ds(base, ECH)])


_d2_call = pl.kernel(
    _d2_body,
    out_type=jax.ShapeDtypeStruct((E,), jnp.float32),
    mesh=_mesh,
    scratch_types=[
        pltpu.VMEM((N * 8,), jnp.float32),
        pltpu.VMEM((ECH,), jnp.int32),
        pltpu.VMEM((ECH,), jnp.int32),
        pltpu.VMEM((ECH,), jnp.float32),
    ],
    compiler_params=_sc_params,
)


def _radial_body(d2_ref, w1_ref, b1_ref, w2_ref, b2_ref, w3_ref, b3_ref, o_ref):
    d2 = d2_ref[...]
    dist = jnp.sqrt(d2 + 1e-12)
    h = jnp.maximum(dist * w1_ref[...] + b1_ref[...], 0.0)
    h = jnp.maximum(
        jnp.dot(h, w2_ref[...], preferred_element_type=jnp.float32)
        + b2_ref[...], 0.0)
    w = jnp.dot(h, w3_ref[...], preferred_element_type=jnp.float32) + b3_ref[...]
    w = w * 0.25
    for q in range(Q):
        o_ref[q] = w[:, q * DQ:(q + 1) * DQ]


_radial_call = pl.pallas_call(
    _radial_body,
    grid=(E // EB,),
    in_specs=[
        pl.BlockSpec((EB, 1), lambda i: (i, 0)),
        pl.BlockSpec((1, H), lambda i: (0, 0)),
        pl.BlockSpec((1, H), lambda i: (0, 0)),
        pl.BlockSpec((H, H), lambda i: (0, 0)),
        pl.BlockSpec((1, H), lambda i: (0, 0)),
        pl.BlockSpec((H, D), lambda i: (0, 0)),
        pl.BlockSpec((1, D), lambda i: (0, 0)),
    ],
    out_specs=pl.BlockSpec((Q, EB, DQ), lambda i: (0, i, 0)),
    out_shape=jax.ShapeDtypeStruct((Q, E, DQ), jnp.float32),
)


def _ln_mm_math(x, g, b, wm, ws, bs):
    mu = jnp.mean(x, axis=1, keepdims=True)
    var = jnp.mean((x - mu) * (x - mu), axis=1, keepdims=True)
    h = (x - mu) / jnp.sqrt(var + 1e-5) * g + b
    xm = jnp.dot(h, wm, preferred_element_type=jnp.float32)
    xs = jnp.dot(h, ws, preferred_element_type=jnp.float32) + bs
    return xm, xs


def _ln_mm_body(x_ref, g_ref, b_ref, wm_ref, ws_ref, bs_ref, xm_ref, xs_ref):
    xm, xs = _ln_mm_math(x_ref[...], g_ref[...], b_ref[...],
                         wm_ref[...], ws_ref[...], bs_ref[...])
    for q in range(Q):
        xm_ref[q] = xm[:, q * DQ:(q + 1) * DQ]
    xs_ref[...] = xs


def _combine_ln_mm_body(xs_in_ref, agg_ref, skip_ref, g_ref, b_ref,
                        wm_ref, ws_ref, bs_ref, feat_ref, xm_ref, xs_ref):
    a = agg_ref[...]
    conv = xs_in_ref[...] + jnp.concatenate([a[q] for q in range(Q)], axis=1)
    feat = jnp.maximum(conv, 0.0) + skip_ref[...]
    feat_ref[...] = feat
    xm, xs = _ln_mm_math(feat, g_ref[...], b_ref[...],
                         wm_ref[...], ws_ref[...], bs_ref[...])
    for q in range(Q):
        xm_ref[q] = xm[:, q * DQ:(q + 1) * DQ]
    xs_ref[...] = xs


def _final_body(xs_in_ref, agg_ref, g_ref, b_ref, o_ref):
    a = agg_ref[...]
    x = xs_in_ref[...] + jnp.concatenate([a[q] for q in range(Q)], axis=1)
    mu = jnp.mean(x, axis=1, keepdims=True)
    var = jnp.mean((x - mu) * (x - mu), axis=1, keepdims=True)
    o_ref[...] = (x - mu) / jnp.sqrt(var + 1e-5) * g_ref[...] + b_ref[...]


_vec_spec = pl.BlockSpec((1, D), lambda i: (0, 0))
_mat_spec = pl.BlockSpec((D, D), lambda i: (0, 0))
_row_spec = pl.BlockSpec((NB, D), lambda i: (i, 0))
_quarter_spec = pl.BlockSpec((Q, NB, DQ), lambda i: (0, i, 0))

_ln_mm_call = pl.pallas_call(
    _ln_mm_body,
    grid=(N // NB,),
    in_specs=[_row_spec, _vec_spec, _vec_spec, _mat_spec, _mat_spec, _vec_spec],
    out_specs=[_quarter_spec, _row_spec],
    out_shape=[jax.ShapeDtypeStruct((Q, N, DQ), jnp.float32),
               jax.ShapeDtypeStruct((N, D), jnp.float32)],
)

_combine_ln_mm_call = pl.pallas_call(
    _combine_ln_mm_body,
    grid=(N // NB,),
    in_specs=[_row_spec, _quarter_spec, _row_spec,
              _vec_spec, _vec_spec, _mat_spec, _mat_spec, _vec_spec],
    out_specs=[_row_spec, _quarter_spec, _row_spec],
    out_shape=[jax.ShapeDtypeStruct((N, D), jnp.float32),
               jax.ShapeDtypeStruct((Q, N, DQ), jnp.float32),
               jax.ShapeDtypeStruct((N, D), jnp.float32)],
)

_final_call = pl.pallas_call(
    _final_body,
    grid=(N // NB,),
    in_specs=[_row_spec, _quarter_spec, _vec_spec, _vec_spec],
    out_specs=_row_spec,
    out_shape=jax.ShapeDtypeStruct((N, D), jnp.float32),
)


def _msg_body(xm_hbm, w_hbm, src3_hbm, dst3_hbm, out_hbm,
              src_v, dst_v, wb_a, wb_b, rows_a, rows_b, prod_a, prod_b,
              zbuf, acc, sem_ga, sem_gb, sem_wa, sem_wb, sem_sa, sem_sb):
    c = lax.axis_index("c")
    s = lax.axis_index("s")
    pltpu.sync_copy(src3_hbm.at[s], src_v)
    pltpu.sync_copy(dst3_hbm.at[s], dst_v)

    def zb_body(r, carry):
        zero16 = jnp.zeros((16,), jnp.float32)
        for rr in range(4):
            for k in range(DQ // 16):
                zbuf[r * 4 + rr, pl.ds(k * 16, 16)] = zero16
        return carry

    lax.fori_loop(0, BLK // 4, zb_body, 0)

    def shift(delta):
        def off_body(r, carry):
            for k in range(BLK // 16):
                sl = src_v[r, pl.ds(k * 16, 16)]
                src_v[r, pl.ds(k * 16, 16)] = sl + delta
            return carry

        lax.fori_loop(0, NBLK, off_body, 0)

    def issue_in(j, rows, wb, sem_g, sem_w, ebase):
        pltpu.async_copy(xm_hbm.at[src_v.at[j]], rows, sem_g)
        pltpu.async_copy(w_hbm.at[pl.ds(ebase + j * BLK, BLK)], wb, sem_w)

    def wait_in(j, rows, wb, sem_g, sem_w, ebase):
        pltpu.make_async_copy(xm_hbm.at[src_v.at[j]], rows, sem_g).wait()
        pltpu.make_async_copy(
            w_hbm.at[pl.ds(ebase + j * BLK, BLK)], wb, sem_w).wait()

    def mul(rows, wb, prod):
        def mul_body(r, carry2):
            for rr in range(4):
                for k in range(DQ // 16):
                    v = (rows[r * 4 + rr, pl.ds(k * 16, 16)]
                         * wb[r * 4 + rr, pl.ds(k * 16, 16)])
                    prod[r * 4 + rr, pl.ds(k * 16, 16)] = v
            return carry2

        lax.fori_loop(0, BLK // 4, mul_body, 0)

    def scat_issue(j, prod, sem_s):
        pltpu.async_copy(prod, acc.at[dst_v.at[j]], sem_s, add=True)

    def scat_wait(j, prod, sem_s):
        pltpu.make_async_copy(prod, acc.at[dst_v.at[j]], sem_s).wait()

    shift((2 * c * N).astype(jnp.int32))
    for qq in range(2):
        if qq:
            shift(jnp.int32(N))
        f = 2 * c + qq
        for z in range(ROWS_PER // BLK):
            pltpu.sync_copy(zbuf, acc.at[pl.ds(s * ROWS_PER + z * BLK, BLK)])
        plsc.subcore_barrier()
        ebase = f * E + s * EC

        issue_in(0, rows_a, wb_a, sem_ga, sem_wa, ebase)

        def pair_body(i, carry):
            ja = 2 * i
            jb = 2 * i + 1

            @pl.when(i > 0)
            def _():
                scat_wait(ja - 2, prod_a, sem_sa)

            issue_in(jb, rows_b, wb_b, sem_gb, sem_wb, ebase)
            wait_in(ja, rows_a, wb_a, sem_ga, sem_wa, ebase)
            mul(rows_a, wb_a, prod_a)
            scat_issue(ja, prod_a, sem_sa)

            @pl.when(i > 0)
            def _():
                scat_wait(jb - 2, prod_b, sem_sb)

            issue_in(jb + 1, rows_a, wb_a, sem_ga, sem_wa, ebase)
            wait_in(jb, rows_b, wb_b, sem_gb, sem_wb, ebase)
            mul(rows_b, wb_b, prod_b)
            scat_issue(jb, prod_b, sem_sb)
            return carry

        lax.fori_loop(0, (NBLK - 1) // 2, pair_body, 0)

        jl = NBLK - 1
        scat_wait(jl - 2, prod_a, sem_sa)
        wait_in(jl, rows_a, wb_a, sem_ga, sem_wa, ebase)
        mul(rows_a, wb_a, prod_a)
        scat_issue(jl, prod_a, sem_sa)
        scat_wait(jl - 1, prod_b, sem_sb)
        scat_wait(jl, prod_a, sem_sa)
        plsc.subcore_barrier()
        pltpu.sync_copy(acc.at[pl.ds(s * ROWS_PER, ROWS_PER)],
                        out_hbm.at[pl.ds(f * NP + s * ROWS_PER, ROWS_PER)])


_msg_call = pl.kernel(
    _msg_body,
    out_type=jax.ShapeDtypeStruct((Q * NP, DQ), jnp.float32),
    mesh=_mesh,
    scratch_types=[
        pltpu.VMEM((NBLK, BLK), jnp.int32),
        pltpu.VMEM((NBLK, BLK), jnp.int32),
        pltpu.VMEM((BLK, DQ), jnp.float32),
        pltpu.VMEM((BLK, DQ), jnp.float32),
        pltpu.VMEM((BLK, DQ), jnp.float32),
        pltpu.VMEM((BLK, DQ), jnp.float32),
        pltpu.VMEM((BLK, DQ), jnp.float32),
        pltpu.VMEM((BLK, DQ), jnp.float32),
        pltpu.VMEM((BLK, DQ), jnp.float32),
        pltpu.VMEM_SHARED((NP, DQ), jnp.float32),
        pltpu.SemaphoreType.DMA,
        pltpu.SemaphoreType.DMA,
        pltpu.SemaphoreType.DMA,
        pltpu.SemaphoreType.DMA,
        pltpu.SemaphoreType.DMA,
        pltpu.SemaphoreType.DMA,
    ],
    compiler_params=_sc_params,
)


def _row(v):
    return v.reshape(1, -1)


def kernel(x, pos, edge_index, params):
    src = edge_index[0]
    dst = edge_index[1]
    pos8 = jnp.concatenate(
        [pos, jnp.zeros((N, 5), jnp.float32)], axis=1).reshape(-1)
    src3 = src.reshape(NS, NBLK, BLK)
    dst3 = dst.reshape(NS, NBLK, BLK)
    zero_full = jnp.zeros((N, D), jnp.float32)

    d2 = _d2_call(pos8, src, dst)
    d2c = d2.reshape(E, 1)

    def radial(p):
        return _radial_call(d2c, p["mlp_w1"], _row(p["mlp_b1"]),
                            p["mlp_w2"], _row(p["mlp_b2"]),
                            p["mlp_w3"], _row(p["mlp_b3"]))

    p0 = params["layer0"]
    ps = params["layer_s"]
    p1 = params["layer1"]
    wgt0f = radial(p0).reshape(Q * E, DQ)
    wgtsf = radial(ps).reshape(Q * E, DQ)
    wgt1f = radial(p1).reshape(Q * E, DQ)

    def agg(xm, wf):
        out = _msg_call(xm.reshape(Q * N, DQ), wf, src3, dst3)
        return out.reshape(Q, NP, DQ)

    ln0 = params["ln0"]
    ln1 = params["ln1"]
    ln2 = params["ln2"]

    xm, xs = _ln_mm_call(x, _row(ln0["gamma"]), _row(ln0["beta"]),
                         p0["wmsg"], p0["wself"], _row(p0["bself"]))
    a = agg(xm, wgt0f)

    feat = zero_full
    for _ in range(4):
        feat, xm, xs = _combine_ln_mm_call(
            xs, a, feat, _row(ln1["gamma"]), _row(ln1["beta"]),
            ps["wmsg"], ps["wself"], _row(ps["bself"]))
        a = agg(xm, wgtsf)

    feat, xm, xs = _combine_ln_mm_call(
        xs, a, feat, _row(ln1["gamma"]), _row(ln1["beta"]),
        p1["wmsg"], p1["wself"], _row(p1["bself"]))
    a = agg(xm, wgt1f)

    return _final_call(xs, a, _row(ln2["gamma"]), _row(ln2["beta"]))


# speedup vs baseline: 4.3311x; 1.3896x over previous
import functools

import jax
import jax.numpy as jnp
from jax import lax
from jax.experimental import pallas as pl
from jax.experimental.pallas import tpu as pltpu
from jax.experimental.pallas import tpu_sc as plsc

N = 10000
E = 160000
D = 256
Q = 4
DQ = 64
H = 64

NC = 2
NS = 16
NW = NC * NS

ECH = E // NW
EC = E // NS
BLK = 80
NBLK = EC // BLK
NP = 10240
ROWS_PER = NP // NS

NB = 1000
EB = 3200

_mesh = plsc.VectorSubcoreMesh(core_axis_name="c", subcore_axis_name="s")
_sc_params = pltpu.CompilerParams(needs_layout_passes=False,
                                  use_tc_tiling_on_sc=False)


def _d2_body(pos_hbm, src_hbm, dst_hbm, d2_hbm, pos_v, src_v, dst_v, out_v):
    c = lax.axis_index("c")
    s = lax.axis_index("s")
    wid = s * NC + c
    base = wid * ECH
    pltpu.sync_copy(pos_hbm, pos_v)
    pltpu.sync_copy(src_hbm.at[pl.ds(base, ECH)], src_v)
    pltpu.sync_copy(dst_hbm.at[pl.ds(base, ECH)], dst_v)

    def body(i, carry):
        off = jnp.minimum(i * 16, ECH - 16)
        si = src_v[pl.ds(off, 16)] * 8
        di = dst_v[pl.ds(off, 16)] * 8
        acc = jnp.zeros((16,), jnp.float32)
        for cdim in range(3):
            a = plsc.load_gather(pos_v, [si + cdim])
            b = plsc.load_gather(pos_v, [di + cdim])
            dd = b - a
            acc = acc + dd * dd
        out_v[pl.ds(off, 16)] = acc
        return carry

    lax.fori_loop(0, (ECH + 15) // 16, body, 0)
    pltpu.sync_copy(out_v, d2_hbm.at[pl.ds(base, ECH)])


_d2_call = pl.kernel(
    _d2_body,
    out_type=jax.ShapeDtypeStruct((E,), jnp.float32),
    mesh=_mesh,
    scratch_types=[
        pltpu.VMEM((N * 8,), jnp.float32),
        pltpu.VMEM((ECH,), jnp.int32),
        pltpu.VMEM((ECH,), jnp.int32),
        pltpu.VMEM((ECH,), jnp.float32),
    ],
    compiler_params=_sc_params,
)


def _radial_body(d2_ref, w1_ref, b1_ref, w2_ref, b2_ref, w3_ref, b3_ref, o_ref):
    d2 = d2_ref[...]
    dist = jnp.sqrt(d2 + 1e-12)
    h = jnp.maximum(dist * w1_ref[...] + b1_ref[...], 0.0)
    h = jnp.maximum(
        jnp.dot(h, w2_ref[...], preferred_element_type=jnp.float32)
        + b2_ref[...], 0.0)
    w = jnp.dot(h, w3_ref[...], preferred_element_type=jnp.float32) + b3_ref[...]
    w = w * 0.25
    o_ref[0][...] = w[:, :D // 2]
    o_ref[1][...] = w[:, D // 2:]


_radial_call = pl.pallas_call(
    lambda d2, w1, b1, w2, b2, w3, b3, o0, o1: _radial_body(
        d2, w1, b1, w2, b2, w3, b3, (o0, o1)),
    grid=(E // EB,),
    in_specs=[
        pl.BlockSpec((EB, 1), lambda i: (i, 0)),
        pl.BlockSpec((1, H), lambda i: (0, 0)),
        pl.BlockSpec((1, H), lambda i: (0, 0)),
        pl.BlockSpec((H, H), lambda i: (0, 0)),
        pl.BlockSpec((1, H), lambda i: (0, 0)),
        pl.BlockSpec((H, D), lambda i: (0, 0)),
        pl.BlockSpec((1, D), lambda i: (0, 0)),
    ],
    out_specs=[pl.BlockSpec((EB, D // 2), lambda i: (i, 0)),
               pl.BlockSpec((EB, D // 2), lambda i: (i, 0))],
    out_shape=[jax.ShapeDtypeStruct((E, D // 2), jnp.float32),
               jax.ShapeDtypeStruct((E, D // 2), jnp.float32)],
)


def _ln_mm_math(x, g, b, wm, ws, bs):
    mu = jnp.mean(x, axis=1, keepdims=True)
    var = jnp.mean((x - mu) * (x - mu), axis=1, keepdims=True)
    h = (x - mu) / jnp.sqrt(var + 1e-5) * g + b
    xm = jnp.dot(h, wm, preferred_element_type=jnp.float32)
    xs = jnp.dot(h, ws, preferred_element_type=jnp.float32) + bs
    return xm, xs


def _store_xm_quarters(xm_ref, xm):
    for q in range(Q):
        xm_ref[q] = xm[:, q * DQ:(q + 1) * DQ]


def _ln_mm_body(x_ref, g_ref, b_ref, wm_ref, ws_ref, bs_ref, xm_ref, xs_ref):
    xm, xs = _ln_mm_math(x_ref[...], g_ref[...], b_ref[...],
                         wm_ref[...], ws_ref[...], bs_ref[...])
    _store_xm_quarters(xm_ref, xm)
    xs_ref[...] = xs


def _combine_ln_mm_body(xs_in_ref, a0_ref, a1_ref, skip_ref, g_ref, b_ref,
                        wm_ref, ws_ref, bs_ref, feat_ref, xm_ref, xs_ref):
    agg = jnp.concatenate([a0_ref[...], a1_ref[...]], axis=1)
    conv = xs_in_ref[...] + agg
    feat = jnp.maximum(conv, 0.0) + skip_ref[...]
    feat_ref[...] = feat
    xm, xs = _ln_mm_math(feat, g_ref[...], b_ref[...],
                         wm_ref[...], ws_ref[...], bs_ref[...])
    _store_xm_quarters(xm_ref, xm)
    xs_ref[...] = xs


def _final_body(xs_in_ref, a0_ref, a1_ref, g_ref, b_ref, o_ref):
    agg = jnp.concatenate([a0_ref[...], a1_ref[...]], axis=1)
    x = xs_in_ref[...] + agg
    mu = jnp.mean(x, axis=1, keepdims=True)
    var = jnp.mean((x - mu) * (x - mu), axis=1, keepdims=True)
    o_ref[...] = (x - mu) / jnp.sqrt(var + 1e-5) * g_ref[...] + b_ref[...]


_vec_spec = pl.BlockSpec((1, D), lambda i: (0, 0))
_mat_spec = pl.BlockSpec((D, D), lambda i: (0, 0))
_row_spec = pl.BlockSpec((NB, D), lambda i: (i, 0))
_half_spec = pl.BlockSpec((NB, D // 2), lambda i: (i, 0))
_quarter_spec = pl.BlockSpec((Q, NB, DQ), lambda i: (0, i, 0))

_ln_mm_call = pl.pallas_call(
    _ln_mm_body,
    grid=(N // NB,),
    in_specs=[_row_spec, _vec_spec, _vec_spec, _mat_spec, _mat_spec, _vec_spec],
    out_specs=[_quarter_spec, _row_spec],
    out_shape=[jax.ShapeDtypeStruct((Q, N, DQ), jnp.float32),
               jax.ShapeDtypeStruct((N, D), jnp.float32)],
)

_combine_ln_mm_call = pl.pallas_call(
    _combine_ln_mm_body,
    grid=(N // NB,),
    in_specs=[_row_spec, _half_spec, _half_spec, _row_spec,
              _vec_spec, _vec_spec, _mat_spec, _mat_spec, _vec_spec],
    out_specs=[_row_spec, _quarter_spec, _row_spec],
    out_shape=[jax.ShapeDtypeStruct((N, D), jnp.float32),
               jax.ShapeDtypeStruct((Q, N, DQ), jnp.float32),
               jax.ShapeDtypeStruct((N, D), jnp.float32)],
)

_final_call = pl.pallas_call(
    _final_body,
    grid=(N // NB,),
    in_specs=[_row_spec, _half_spec, _half_spec, _vec_spec, _vec_spec],
    out_specs=_row_spec,
    out_shape=jax.ShapeDtypeStruct((N, D), jnp.float32),
)


def _msg_body(xm_hbm, w0_hbm, w1_hbm, src3_hbm, dst3_hbm, out0_hbm, out1_hbm,
              src_v, dst_v, wb_a, wb_b, rows_a, rows_b, prod_a, prod_b,
              zbuf, acc, sem_ga, sem_gb, sem_wa, sem_wb):
    c = lax.axis_index("c")
    s = lax.axis_index("s")
    pltpu.sync_copy(src3_hbm.at[s], src_v)
    pltpu.sync_copy(dst3_hbm.at[s], dst_v)

    def zb_body(r, carry):
        zero16 = jnp.zeros((16,), jnp.float32)
        for rr in range(4):
            for k in range(DQ // 16):
                zbuf[r * 4 + rr, pl.ds(k * 16, 16)] = zero16
        return carry

    lax.fori_loop(0, BLK // 4, zb_body, 0)

    def shift(delta):
        def off_body(r, carry):
            for k in range(BLK // 16):
                sl = src_v[r, pl.ds(k * 16, 16)]
                src_v[r, pl.ds(k * 16, 16)] = sl + delta
            return carry

        lax.fori_loop(0, NBLK, off_body, 0)

    ebase = s * EC
    ccol = c * DQ

    def mul(rows, wb, prod):
        def mul_body(r, carry2):
            for rr in range(4):
                for k in range(DQ // 16):
                    v = (rows[r * 4 + rr, pl.ds(k * 16, 16)]
                         * wb[r * 4 + rr, pl.ds(k * 16, 16)])
                    prod[r * 4 + rr, pl.ds(k * 16, 16)] = v
            return carry2

        lax.fori_loop(0, BLK // 4, mul_body, 0)

    def scat(j, prod):
        pltpu.sync_copy(prod, acc.at[dst_v.at[j]], add=True)

    shift(c * N)
    for qq in range(2):
        if qq:
            shift(jnp.int32(2 * N))
        w_hbm = (w0_hbm, w1_hbm)[qq]
        out_hbm = (out0_hbm, out1_hbm)[qq]

        def issue_in(j, rows, wb, sem_g, sem_w):
            pltpu.async_copy(xm_hbm.at[src_v.at[j]], rows, sem_g)
            pltpu.async_copy(
                w_hbm.at[pl.ds(ebase + j * BLK, BLK), pl.ds(ccol, DQ)],
                wb, sem_w)

        def wait_in(j, rows, wb, sem_g, sem_w):
            pltpu.make_async_copy(xm_hbm.at[src_v.at[j]], rows, sem_g).wait()
            pltpu.make_async_copy(
                w_hbm.at[pl.ds(ebase + j * BLK, BLK), pl.ds(ccol, DQ)],
                wb, sem_w).wait()

        for z in range(ROWS_PER // BLK):
            pltpu.sync_copy(zbuf, acc.at[pl.ds(s * ROWS_PER + z * BLK, BLK)])
        plsc.subcore_barrier()

        issue_in(0, rows_a, wb_a, sem_ga, sem_wa)

        def pair_body(i, carry):
            ja = 2 * i
            jb = 2 * i + 1
            issue_in(jb, rows_b, wb_b, sem_gb, sem_wb)
            wait_in(ja, rows_a, wb_a, sem_ga, sem_wa)
            mul(rows_a, wb_a, prod_a)
            scat(ja, prod_a)
            issue_in(jb + 1, rows_a, wb_a, sem_ga, sem_wa)
            wait_in(jb, rows_b, wb_b, sem_gb, sem_wb)
            mul(rows_b, wb_b, prod_b)
            scat(jb, prod_b)
            return carry

        lax.fori_loop(0, (NBLK - 1) // 2, pair_body, 0)

        jl = NBLK - 1
        wait_in(jl, rows_a, wb_a, sem_ga, sem_wa)
        mul(rows_a, wb_a, prod_a)
        scat(jl, prod_a)
        plsc.subcore_barrier()
        pltpu.sync_copy(
            acc.at[pl.ds(s * ROWS_PER, ROWS_PER)],
            out_hbm.at[pl.ds(s * ROWS_PER, ROWS_PER), pl.ds(ccol, DQ)])


_msg_call = pl.kernel(
    _msg_body,
    out_type=[jax.ShapeDtypeStruct((NP, D // 2), jnp.float32),
              jax.ShapeDtypeStruct((NP, D // 2), jnp.float32)],
    mesh=_mesh,
    scratch_types=[
        pltpu.VMEM((NBLK, BLK), jnp.int32),
        pltpu.VMEM((NBLK, BLK), jnp.int32),
        pltpu.VMEM((BLK, DQ), jnp.float32),
        pltpu.VMEM((BLK, DQ), jnp.float32),
        pltpu.VMEM((BLK, DQ), jnp.float32),
        pltpu.VMEM((BLK, DQ), jnp.float32),
        pltpu.VMEM((BLK, DQ), jnp.float32),
        pltpu.VMEM((BLK, DQ), jnp.float32),
        pltpu.VMEM((BLK, DQ), jnp.float32),
        pltpu.VMEM_SHARED((NP, DQ), jnp.float32),
        pltpu.SemaphoreType.DMA,
        pltpu.SemaphoreType.DMA,
        pltpu.SemaphoreType.DMA,
        pltpu.SemaphoreType.DMA,
    ],
    compiler_params=_sc_params,
)


def _row(v):
    return v.reshape(1, -1)


def kernel(x, pos, edge_index, params):
    src = edge_index[0]
    dst = edge_index[1]
    pos8 = jnp.concatenate(
        [pos, jnp.zeros((N, 5), jnp.float32)], axis=1).reshape(-1)
    src3 = src.reshape(NS, NBLK, BLK)
    dst3 = dst.reshape(NS, NBLK, BLK)
    zero_full = jnp.zeros((N, D), jnp.float32)

    d2 = _d2_call(pos8, src, dst)
    d2c = d2.reshape(E, 1)

    def radial(p):
        return _radial_call(d2c, p["mlp_w1"], _row(p["mlp_b1"]),
                            p["mlp_w2"], _row(p["mlp_b2"]),
                            p["mlp_w3"], _row(p["mlp_b3"]))

    p0 = params["layer0"]
    ps = params["layer_s"]
    p1 = params["layer1"]
    wgt0f = radial(p0)
    wgtsf = radial(ps)
    wgt1f = radial(p1)

    def agg(xm, wf):
        return _msg_call(xm.reshape(Q * N, DQ), wf[0], wf[1], src3, dst3)

    ln0 = params["ln0"]
    ln1 = params["ln1"]
    ln2 = params["ln2"]

    xm, xs = _ln_mm_call(x, _row(ln0["gamma"]), _row(ln0["beta"]),
                         p0["wmsg"], p0["wself"], _row(p0["bself"]))
    a = agg(xm, wgt0f)

    feat = zero_full
    for _ in range(4):
        feat, xm, xs = _combine_ln_mm_call(
            xs, a[0], a[1], feat, _row(ln1["gamma"]), _row(ln1["beta"]),
            ps["wmsg"], ps["wself"], _row(ps["bself"]))
        a = agg(xm, wgtsf)

    feat, xm, xs = _combine_ln_mm_call(
        xs, a[0], a[1], feat, _row(ln1["gamma"]), _row(ln1["beta"]),
        p1["wmsg"], p1["wself"], _row(p1["bself"]))
    a = agg(xm, wgt1f)

    return _final_call(xs, a[0], a[1], _row(ln2["gamma"]), _row(ln2["beta"]))
